# baseline (speedup 1.0000x reference)
import jax
import jax.numpy as jnp
from jax import lax
from jax.experimental import pallas as pl
from jax.experimental.pallas import tpu as pltpu
from jax.experimental.pallas import tpu_sc as plsc

N_PMTS = 2330
B_GRAPHS = 32
N = B_GRAPHS * N_PMTS
E = N * 16
CH = 128
W_AGG = 8
GK = 16
CORE0_SHARE_PCT = 64
ROW_BLK = 1864


def _sc_worker_geom(nc, ns):
    nw = nc * ns
    k = -(-E // (nw * CH))
    k = -(-k // GK) * GK
    zchunks = -(-(N + 1) // (ns * CH))
    sub_rows = zchunks * CH
    acc_rows = ns * sub_rows
    return nw, k, zchunks, sub_rows, acc_rows


def _sc_degree(edges3, zeros1, nc, ns):
    nw, k_chunks, zchunks, sub_rows, acc_rows = _sc_worker_geom(nc, ns)
    mesh = plsc.VectorSubcoreMesh(core_axis_name="c", subcore_axis_name="s",
                                  num_cores=nc, num_subcores=ns)

    def body(e_hbm, z_hbm, out_hbm, dst_idx, ones_v, acc):
        cid = lax.axis_index("c")
        sid = lax.axis_index("s")
        t = cid * ns + sid
        pltpu.sync_copy(e_hbm.at[1, pl.ds(t * k_chunks, k_chunks), :], dst_idx)
        for i in range(CH // 16):
            ones_v[pl.ds(i * 16, 16)] = jnp.ones((16,), jnp.float32)
        base = sid * sub_rows
        pltpu.sync_copy(z_hbm, acc.at[pl.ds(base, sub_rows)])
        plsc.subcore_barrier()

        def scat(j, _):
            pltpu.sync_copy(ones_v, acc.at[dst_idx.at[j]], add=True)
            return 0
        lax.fori_loop(0, k_chunks, scat, 0)
        plsc.subcore_barrier()
        pltpu.sync_copy(acc.at[pl.ds(base, sub_rows)], out_hbm.at[cid, sid])

    return pl.kernel(
        body,
        out_type=jax.ShapeDtypeStruct((nc, ns, sub_rows), jnp.float32),
        mesh=mesh,
        scratch_types=[
            pltpu.VMEM((k_chunks, CH), jnp.int32),
            pltpu.VMEM((CH,), jnp.float32),
            pltpu.VMEM_SHARED((acc_rows,), jnp.float32),
        ],
        compiler_params=pltpu.CompilerParams(use_tc_tiling_on_sc=False),
    )(edges3, zeros1)


def _sc_aggregate(g, edges3, zeros2, nc, ns):
    nw, k_chunks, zchunks, sub_rows, acc_rows = _sc_worker_geom(nc, ns)
    mesh = plsc.VectorSubcoreMesh(core_axis_name="c", subcore_axis_name="s",
                                  num_cores=nc, num_subcores=ns)

    n_groups = k_chunks // GK
    total_groups = 2 * n_groups
    g_a = (total_groups * CORE0_SHARE_PCT + 50) // 100
    g_b = total_groups - g_a
    c_a, c_b = g_a * GK, g_b * GK
    k_max = max(c_a, c_b)

    def body(g_hbm, e_hbm, z_hbm, out_hbm,
             src_idx, dst_idx, bufs, acc, gsem, ssem, isem):
        cid = lax.axis_index("c")
        sid = lax.axis_index("s")
        base = sid * sub_rows
        pltpu.sync_copy(z_hbm, acc.at[pl.ds(base, sub_rows), :])

        def run(start, n_chk, n_grp):
            def iload_sync(grp):
                bk = grp % 3
                pltpu.sync_copy(e_hbm.at[0, pl.ds(start + grp * GK, GK), :],
                                src_idx.at[bk])
                pltpu.sync_copy(e_hbm.at[1, pl.ds(start + grp * GK, GK), :],
                                dst_idx.at[bk])

            def iload_start(grp):
                bk = grp % 3
                pltpu.async_copy(e_hbm.at[0, pl.ds(start + grp * GK, GK), :],
                                 src_idx.at[bk], isem)
                pltpu.async_copy(e_hbm.at[1, pl.ds(start + grp * GK, GK), :],
                                 dst_idx.at[bk], isem)

            def iload_wait(grp):
                bk = grp % 3
                pltpu.make_async_copy(
                    e_hbm.at[0, pl.ds(start + grp * GK, GK), :],
                    src_idx.at[bk], isem).wait()
                pltpu.make_async_copy(
                    e_hbm.at[1, pl.ds(start + grp * GK, GK), :],
                    dst_idx.at[bk], isem).wait()

            def gstart(grp, b):
                pltpu.async_copy(g_hbm.at[src_idx.at[grp % 3, b]],
                                 bufs.at[(grp % 2) * GK + b], gsem)

            def gwait(grp, b):
                pltpu.make_async_copy(g_hbm.at[src_idx.at[grp % 3, b]],
                                      bufs.at[(grp % 2) * GK + b], gsem).wait()

            def sstart(grp, b):
                pltpu.async_copy(bufs.at[(grp % 2) * GK + b],
                                 acc.at[dst_idx.at[grp % 3, b]], ssem,
                                 add=True)

            def swait(grp, b):
                pltpu.make_async_copy(bufs.at[(grp % 2) * GK + b],
                                      acc.at[dst_idx.at[grp % 3, b]],
                                      ssem).wait()

            iload_sync(0)
            for b in range(GK):
                gstart(0, b)

            @pl.when(1 < n_grp)
            def _():
                iload_start(1)

            def loop(m, _):
                for b in range(GK):
                    gwait(m, b)

                @pl.when(m >= 1)
                def _():
                    for b in range(GK):
                        swait(m - 1, b)
                for b in range(GK):
                    sstart(m, b)

                @pl.when(m + 1 < n_grp)
                def _():
                    iload_wait(m + 1)
                    for b in range(GK):
                        gstart(m + 1, b)

                @pl.when(m + 2 < n_grp)
                def _():
                    iload_start(m + 2)
                return 0
            lax.fori_loop(0, n_grp, loop, 0)

            for b in range(GK):
                swait(n_grp - 1, b)

        @pl.when(cid == 0)
        def _():
            run(sid * c_a, c_a, g_a)

        @pl.when(cid == 1)
        def _():
            run(ns * c_a + sid * c_b, c_b, g_b)

        plsc.subcore_barrier()
        pltpu.sync_copy(acc.at[pl.ds(base, sub_rows), :],
                        out_hbm.at[cid, pl.ds(base, sub_rows), :])

    return pl.kernel(
        body,
        out_type=jax.ShapeDtypeStruct((nc, acc_rows, W_AGG), jnp.float32),
        mesh=mesh,
        scratch_types=[
            pltpu.VMEM((3, GK, CH), jnp.int32),
            pltpu.VMEM((3, GK, CH), jnp.int32),
            pltpu.VMEM((2 * GK, CH, W_AGG), jnp.float32),
            pltpu.VMEM_SHARED((acc_rows, W_AGG), jnp.float32),
            pltpu.SemaphoreType.DMA,
            pltpu.SemaphoreType.DMA,
            pltpu.SemaphoreType.DMA,
        ],
        compiler_params=pltpu.CompilerParams(use_tc_tiling_on_sc=False),
    )(g, edges3, zeros2)


def _tc_prep(degp, xpad):
    grid = (N // ROW_BLK,)

    def body(deg_ref, x_ref, dinv_ref, g1_ref):
        d = deg_ref[0] + deg_ref[1] + 1.0
        dv = lax.rsqrt(d)
        dinv_ref[...] = dv
        xb = x_ref[...].astype(jnp.bfloat16).astype(jnp.float32)
        g1_ref[...] = xb * dv

    return pl.pallas_call(
        body,
        grid=grid,
        in_specs=[
            pl.BlockSpec((2, ROW_BLK, 1), lambda i: (0, i, 0)),
            pl.BlockSpec((ROW_BLK, W_AGG), lambda i: (i, 0)),
        ],
        out_specs=[
            pl.BlockSpec((ROW_BLK, 1), lambda i: (i, 0)),
            pl.BlockSpec((ROW_BLK, W_AGG), lambda i: (i, 0)),
        ],
        out_shape=[
            jax.ShapeDtypeStruct((N, 1), jnp.float32),
            jax.ShapeDtypeStruct((N + 8, W_AGG), jnp.float32),
        ],
    )(degp, xpad)


def _tc_layer12(agg, gprev, dinv, w_a, b_a, w_bst):
    grid = (N // ROW_BLK,)

    def body(agg_ref, g_ref, dinv_ref, wa_ref, ba_ref, wb_ref, out_ref):
        dv = dinv_ref[...]
        s = (agg_ref[0] + agg_ref[1] + g_ref[...]) * dv
        h = jnp.dot(s, wa_ref[...], preferred_element_type=jnp.float32, precision=lax.Precision.HIGHEST)
        h = jnp.maximum(h + ba_ref[...], 0.0).astype(jnp.bfloat16)
        out_ref[0] = jnp.dot(h, wb_ref[0], preferred_element_type=jnp.float32) * dv
        out_ref[1] = jnp.dot(h, wb_ref[1], preferred_element_type=jnp.float32) * dv

    return pl.pallas_call(
        body,
        grid=grid,
        in_specs=[
            pl.BlockSpec((2, ROW_BLK, W_AGG), lambda i: (0, i, 0)),
            pl.BlockSpec((ROW_BLK, W_AGG), lambda i: (i, 0)),
            pl.BlockSpec((ROW_BLK, 1), lambda i: (i, 0)),
            pl.BlockSpec(w_a.shape, lambda i: (0, 0)),
            pl.BlockSpec(b_a.shape, lambda i: (0, 0)),
            pl.BlockSpec(w_bst.shape, lambda i: (0, 0, 0)),
        ],
        out_specs=pl.BlockSpec((2, ROW_BLK, W_AGG), lambda i: (0, i, 0)),
        out_shape=jax.ShapeDtypeStruct((2, N + 8, W_AGG), jnp.float32),
    )(agg, gprev, dinv, w_a, b_a, w_bst)


def _tc_layer3(agg_a, agg_b, g2, dinv, b2a, b2b, w3a, w3b):
    grid = (N // ROW_BLK,)

    def body(aa_ref, ab_ref, g_ref, dinv_ref, ba_ref, bb_ref,
             wa_ref, wb_ref, out_ref):
        dv = dinv_ref[...]
        sa = (aa_ref[0] + aa_ref[1] + g_ref[0]) * dv
        sb = (ab_ref[0] + ab_ref[1] + g_ref[1]) * dv
        ha = jnp.maximum(sa + ba_ref[...], 0.0).astype(jnp.bfloat16)
        hb = jnp.maximum(sb + bb_ref[...], 0.0).astype(jnp.bfloat16)
        o = (jnp.dot(ha, wa_ref[...], preferred_element_type=jnp.float32)
             + jnp.dot(hb, wb_ref[...], preferred_element_type=jnp.float32))
        out_ref[...] = o * dv

    return pl.pallas_call(
        body,
        grid=grid,
        in_specs=[
            pl.BlockSpec((2, ROW_BLK, W_AGG), lambda i: (0, i, 0)),
            pl.BlockSpec((2, ROW_BLK, W_AGG), lambda i: (0, i, 0)),
            pl.BlockSpec((2, ROW_BLK, W_AGG), lambda i: (0, i, 0)),
            pl.BlockSpec((ROW_BLK, 1), lambda i: (i, 0)),
            pl.BlockSpec(b2a.shape, lambda i: (0, 0)),
            pl.BlockSpec(b2b.shape, lambda i: (0, 0)),
            pl.BlockSpec(w3a.shape, lambda i: (0, 0)),
            pl.BlockSpec(w3b.shape, lambda i: (0, 0)),
        ],
        out_specs=pl.BlockSpec((ROW_BLK, W_AGG), lambda i: (i, 0)),
        out_shape=jax.ShapeDtypeStruct((N + 8, W_AGG), jnp.float32),
    )(agg_a, agg_b, g2, dinv, b2a, b2b, w3a, w3b)


def _tc_final_nodes(agg, gprev, dinv, b_a):
    grid = (N // ROW_BLK,)

    def body(agg_ref, g_ref, dinv_ref, ba_ref, out_ref):
        s = (agg_ref[0] + agg_ref[1] + g_ref[...]) * dinv_ref[...]
        out_ref[...] = jnp.maximum(s + ba_ref[...], 0.0)

    return pl.pallas_call(
        body,
        grid=grid,
        in_specs=[
            pl.BlockSpec((2, ROW_BLK, W_AGG), lambda i: (0, i, 0)),
            pl.BlockSpec((ROW_BLK, W_AGG), lambda i: (i, 0)),
            pl.BlockSpec((ROW_BLK, 1), lambda i: (i, 0)),
            pl.BlockSpec(b_a.shape, lambda i: (0, 0)),
        ],
        out_specs=pl.BlockSpec((ROW_BLK, W_AGG), lambda i: (i, 0)),
        out_shape=jax.ShapeDtypeStruct((N, W_AGG), jnp.float32),
    )(agg, gprev, dinv, b_a)


def _tc_readout(h3r, wl1p, bl1, gamma, beta, wl2, bl2):
    def body(h_ref, w1_ref, b1_ref, ga_ref, be_ref, w2_ref, b2_ref, out_ref):
        z = jnp.dot(h_ref[...].astype(jnp.bfloat16), w1_ref[...],
                    preferred_element_type=jnp.float32)
        z = z + b1_ref[...]
        mean = jnp.mean(z, axis=0, keepdims=True)
        var = jnp.mean((z - mean) ** 2, axis=0, keepdims=True)
        zn = (z - mean) * lax.rsqrt(var + 1e-5) * ga_ref[...] + be_ref[...]
        zr = jnp.maximum(zn, 0.0)
        o = jnp.dot(zr.astype(jnp.bfloat16), w2_ref[...],
                    preferred_element_type=jnp.float32)
        out_ref[...] = o + b2_ref[...]

    return pl.pallas_call(
        body,
        out_shape=jax.ShapeDtypeStruct((B_GRAPHS, 1), jnp.float32),
    )(h3r, wl1p, bl1, gamma, beta, wl2, bl2)


def kernel(x, edge_index, W1, b1, W2, b2, W3, b3, Wl1, bl1, gamma, beta,
           Wl2, bl2):
    try:
        info = plsc.get_sparse_core_info()
        nc, ns = info.num_cores, info.num_subcores
    except Exception:
        nc, ns = 2, 16
    nw, k_chunks, zchunks, sub_rows, acc_rows = _sc_worker_geom(nc, ns)

    epad = nw * k_chunks * CH - E
    edges3 = jnp.pad(edge_index, ((0, 0), (0, epad)),
                     constant_values=N).reshape(2, nw * k_chunks, CH)

    zeros1 = jnp.zeros((sub_rows,), jnp.float32)
    zeros2 = jnp.zeros((sub_rows, W_AGG), jnp.float32)

    degp = _sc_degree(edges3, zeros1, nc, ns)
    degp = degp.reshape(nc, ns * sub_rows)[:, :N, None]

    bf = jnp.bfloat16
    w1p = jnp.pad(W1, ((0, W_AGG - 5), (0, 0))).astype(bf).astype(jnp.float32)
    w2st = jnp.stack([W2[:, :W_AGG], W2[:, W_AGG:]]).astype(bf)
    w3p = jnp.pad(W3, ((0, 0), (0, W_AGG - 3))).astype(bf)
    w3a, w3b = w3p[:W_AGG], w3p[W_AGG:]
    b2a, b2b = b2[None, :W_AGG], b2[None, W_AGG:]
    b3p = jnp.pad(b3, (0, W_AGG - 3))[None, :]
    xpad = jnp.pad(x, ((0, 0), (0, W_AGG - 5)))

    dinv, g1 = _tc_prep(degp, xpad)

    def run_agg(g):
        return _sc_aggregate(g, edges3, zeros2, nc, ns)

    agg1 = run_agg(g1)
    g2 = _tc_layer12(agg1, g1, dinv, w1p, b1[None, :], w2st)
    agg2a = run_agg(g2[0])
    agg2b = run_agg(g2[1])
    g3 = _tc_layer3(agg2a, agg2b, g2, dinv, b2a, b2b, w3a, w3b)
    agg3 = run_agg(g3)
    h3 = _tc_final_nodes(agg3, g3, dinv, b3p)

    h3r = h3.reshape(B_GRAPHS, N_PMTS * W_AGG)
    wl1p = jnp.pad(Wl1.reshape(N_PMTS, 3, 32),
                   ((0, 0), (0, W_AGG - 3), (0, 0))).reshape(
                       N_PMTS * W_AGG, 32).astype(jnp.bfloat16)
    return _tc_readout(h3r, wl1p, bl1[None, :], gamma[None, :], beta[None, :],
                       Wl2.astype(jnp.bfloat16), bl2[None, :])


# speedup vs baseline: 1.4507x; 1.4507x over previous
import jax
import jax.numpy as jnp
from jax import lax
from jax.experimental import pallas as pl
from jax.experimental.pallas import tpu as pltpu
from jax.experimental.pallas import tpu_sc as plsc

N_PMTS = 2330
B_GRAPHS = 32
N = B_GRAPHS * N_PMTS
E = N * 16
CH = 128
W_AGG = 8
GK = 8
CORE0_SHARE_PCT = 64
ROW_BLK = 1864


def _sc_worker_geom(nc, ns):
    nw = nc * ns
    k = -(-E // (nw * CH))
    k = -(-k // GK) * GK
    zchunks = -(-(N + 1) // (ns * CH))
    sub_rows = zchunks * CH
    acc_rows = ns * sub_rows
    return nw, k, zchunks, sub_rows, acc_rows


def _sc_degree(edges3, zeros1, nc, ns):
    nw, k_chunks, zchunks, sub_rows, acc_rows = _sc_worker_geom(nc, ns)
    mesh = plsc.VectorSubcoreMesh(core_axis_name="c", subcore_axis_name="s",
                                  num_cores=nc, num_subcores=ns)

    def body(e_hbm, z_hbm, out_hbm, dst_idx, ones_v, acc):
        cid = lax.axis_index("c")
        sid = lax.axis_index("s")
        t = cid * ns + sid
        pltpu.sync_copy(e_hbm.at[1, pl.ds(t * k_chunks, k_chunks), :], dst_idx)
        for i in range(CH // 16):
            ones_v[pl.ds(i * 16, 16)] = jnp.ones((16,), jnp.float32)
        base = sid * sub_rows
        pltpu.sync_copy(z_hbm, acc.at[pl.ds(base, sub_rows)])
        plsc.subcore_barrier()

        def scat(j, _):
            pltpu.sync_copy(ones_v, acc.at[dst_idx.at[j]], add=True)
            return 0
        lax.fori_loop(0, k_chunks, scat, 0)
        plsc.subcore_barrier()
        pltpu.sync_copy(acc.at[pl.ds(base, sub_rows)], out_hbm.at[cid, sid])

    return pl.kernel(
        body,
        out_type=jax.ShapeDtypeStruct((nc, ns, sub_rows), jnp.float32),
        mesh=mesh,
        scratch_types=[
            pltpu.VMEM((k_chunks, CH), jnp.int32),
            pltpu.VMEM((CH,), jnp.float32),
            pltpu.VMEM_SHARED((acc_rows,), jnp.float32),
        ],
        compiler_params=pltpu.CompilerParams(use_tc_tiling_on_sc=False),
    )(edges3, zeros1)


def _sc_aggregate(g, edges3, zeros2, nc, ns):
    nw, k_chunks, zchunks, sub_rows, acc_rows = _sc_worker_geom(nc, ns)
    mesh = plsc.VectorSubcoreMesh(core_axis_name="c", subcore_axis_name="s",
                                  num_cores=nc, num_subcores=ns)

    n_groups = k_chunks // GK
    total_groups = 2 * n_groups
    g_a = (total_groups * CORE0_SHARE_PCT + 50) // 100
    g_b = total_groups - g_a
    c_a, c_b = g_a * GK, g_b * GK
    k_max = max(c_a, c_b)

    def body(g_hbm, e_hbm, z_hbm, out_hbm,
             src_idx, dst_idx, bufs, acc, gsem, ssem, isem):
        cid = lax.axis_index("c")
        sid = lax.axis_index("s")
        base = sid * sub_rows
        pltpu.sync_copy(z_hbm, acc.at[pl.ds(base, sub_rows), :])

        def run(start, n_chk, n_grp):
            def iload_sync(grp):
                bk = grp % 3
                pltpu.sync_copy(e_hbm.at[0, pl.ds(start + grp * GK, GK), :],
                                src_idx.at[bk])
                pltpu.sync_copy(e_hbm.at[1, pl.ds(start + grp * GK, GK), :],
                                dst_idx.at[bk])

            def iload_start(grp):
                bk = grp % 3
                pltpu.async_copy(e_hbm.at[0, pl.ds(start + grp * GK, GK), :],
                                 src_idx.at[bk], isem)
                pltpu.async_copy(e_hbm.at[1, pl.ds(start + grp * GK, GK), :],
                                 dst_idx.at[bk], isem)

            def iload_wait(grp):
                bk = grp % 3
                pltpu.make_async_copy(
                    e_hbm.at[0, pl.ds(start + grp * GK, GK), :],
                    src_idx.at[bk], isem).wait()
                pltpu.make_async_copy(
                    e_hbm.at[1, pl.ds(start + grp * GK, GK), :],
                    dst_idx.at[bk], isem).wait()

            def gstart(grp, b):
                pltpu.async_copy(g_hbm.at[src_idx.at[grp % 3, b]],
                                 bufs.at[(grp % 2) * GK + b], gsem)

            def gwait(grp, b):
                pltpu.make_async_copy(g_hbm.at[src_idx.at[grp % 3, b]],
                                      bufs.at[(grp % 2) * GK + b], gsem).wait()

            def sstart(grp, b):
                pltpu.async_copy(bufs.at[(grp % 2) * GK + b],
                                 acc.at[dst_idx.at[grp % 3, b]], ssem,
                                 add=True)

            def swait(grp, b):
                pltpu.make_async_copy(bufs.at[(grp % 2) * GK + b],
                                      acc.at[dst_idx.at[grp % 3, b]],
                                      ssem).wait()

            iload_sync(0)
            for b in range(GK):
                gstart(0, b)

            @pl.when(1 < n_grp)
            def _():
                iload_start(1)

            def loop(m, _):
                for b in range(GK):
                    gwait(m, b)

                @pl.when(m >= 1)
                def _():
                    for b in range(GK):
                        swait(m - 1, b)
                for b in range(GK):
                    sstart(m, b)

                @pl.when(m + 1 < n_grp)
                def _():
                    iload_wait(m + 1)
                    for b in range(GK):
                        gstart(m + 1, b)

                @pl.when(m + 2 < n_grp)
                def _():
                    iload_start(m + 2)
                return 0
            lax.fori_loop(0, n_grp, loop, 0)

            for b in range(GK):
                swait(n_grp - 1, b)

        @pl.when(cid == 0)
        def _():
            run(sid * c_a, c_a, g_a)

        @pl.when(cid == 1)
        def _():
            run(ns * c_a + sid * c_b, c_b, g_b)

        plsc.subcore_barrier()
        pltpu.sync_copy(acc.at[pl.ds(base, sub_rows), :],
                        out_hbm.at[cid, pl.ds(base, sub_rows), :])

    return pl.kernel(
        body,
        out_type=jax.ShapeDtypeStruct((nc, acc_rows, W_AGG), jnp.float32),
        mesh=mesh,
        scratch_types=[
            pltpu.VMEM((3, GK, CH), jnp.int32),
            pltpu.VMEM((3, GK, CH), jnp.int32),
            pltpu.VMEM((2 * GK, CH, W_AGG), jnp.float32),
            pltpu.VMEM_SHARED((acc_rows, W_AGG), jnp.float32),
            pltpu.SemaphoreType.DMA,
            pltpu.SemaphoreType.DMA,
            pltpu.SemaphoreType.DMA,
        ],
        compiler_params=pltpu.CompilerParams(use_tc_tiling_on_sc=False),
    )(g, edges3, zeros2)


def _tc_prep(degp, xpad):
    grid = (N // ROW_BLK,)

    def body(deg_ref, x_ref, dinv_ref, g1_ref):
        d = deg_ref[0] + deg_ref[1] + 1.0
        dv = lax.rsqrt(d)
        dinv_ref[...] = dv
        xb = x_ref[...].astype(jnp.bfloat16).astype(jnp.float32)
        g1_ref[...] = xb * dv

    return pl.pallas_call(
        body,
        grid=grid,
        in_specs=[
            pl.BlockSpec((2, ROW_BLK, 1), lambda i: (0, i, 0)),
            pl.BlockSpec((ROW_BLK, W_AGG), lambda i: (i, 0)),
        ],
        out_specs=[
            pl.BlockSpec((ROW_BLK, 1), lambda i: (i, 0)),
            pl.BlockSpec((ROW_BLK, W_AGG), lambda i: (i, 0)),
        ],
        out_shape=[
            jax.ShapeDtypeStruct((N, 1), jnp.float32),
            jax.ShapeDtypeStruct((N + 8, W_AGG), jnp.float32),
        ],
    )(degp, xpad)


def _tc_layer12(agg, gprev, dinv, w_a, b_a, w_bst):
    grid = (N // ROW_BLK,)

    def body(agg_ref, g_ref, dinv_ref, wa_ref, ba_ref, wb_ref, out_ref):
        dv = dinv_ref[...]
        s = (agg_ref[0] + agg_ref[1] + g_ref[...]) * dv
        h = jnp.dot(s, wa_ref[...], preferred_element_type=jnp.float32, precision=lax.Precision.HIGHEST)
        h = jnp.maximum(h + ba_ref[...], 0.0).astype(jnp.bfloat16)
        out_ref[0] = jnp.dot(h, wb_ref[0], preferred_element_type=jnp.float32) * dv
        out_ref[1] = jnp.dot(h, wb_ref[1], preferred_element_type=jnp.float32) * dv

    return pl.pallas_call(
        body,
        grid=grid,
        in_specs=[
            pl.BlockSpec((2, ROW_BLK, W_AGG), lambda i: (0, i, 0)),
            pl.BlockSpec((ROW_BLK, W_AGG), lambda i: (i, 0)),
            pl.BlockSpec((ROW_BLK, 1), lambda i: (i, 0)),
            pl.BlockSpec(w_a.shape, lambda i: (0, 0)),
            pl.BlockSpec(b_a.shape, lambda i: (0, 0)),
            pl.BlockSpec(w_bst.shape, lambda i: (0, 0, 0)),
        ],
        out_specs=pl.BlockSpec((2, ROW_BLK, W_AGG), lambda i: (0, i, 0)),
        out_shape=jax.ShapeDtypeStruct((2, N + 8, W_AGG), jnp.float32),
    )(agg, gprev, dinv, w_a, b_a, w_bst)


def _tc_layer3(agg_a, agg_b, g2, dinv, b2a, b2b, w3a, w3b):
    grid = (N // ROW_BLK,)

    def body(aa_ref, ab_ref, g_ref, dinv_ref, ba_ref, bb_ref,
             wa_ref, wb_ref, out_ref):
        dv = dinv_ref[...]
        sa = (aa_ref[0] + aa_ref[1] + g_ref[0]) * dv
        sb = (ab_ref[0] + ab_ref[1] + g_ref[1]) * dv
        ha = jnp.maximum(sa + ba_ref[...], 0.0).astype(jnp.bfloat16)
        hb = jnp.maximum(sb + bb_ref[...], 0.0).astype(jnp.bfloat16)
        o = (jnp.dot(ha, wa_ref[...], preferred_element_type=jnp.float32)
             + jnp.dot(hb, wb_ref[...], preferred_element_type=jnp.float32))
        out_ref[...] = o * dv

    return pl.pallas_call(
        body,
        grid=grid,
        in_specs=[
            pl.BlockSpec((2, ROW_BLK, W_AGG), lambda i: (0, i, 0)),
            pl.BlockSpec((2, ROW_BLK, W_AGG), lambda i: (0, i, 0)),
            pl.BlockSpec((2, ROW_BLK, W_AGG), lambda i: (0, i, 0)),
            pl.BlockSpec((ROW_BLK, 1), lambda i: (i, 0)),
            pl.BlockSpec(b2a.shape, lambda i: (0, 0)),
            pl.BlockSpec(b2b.shape, lambda i: (0, 0)),
            pl.BlockSpec(w3a.shape, lambda i: (0, 0)),
            pl.BlockSpec(w3b.shape, lambda i: (0, 0)),
        ],
        out_specs=pl.BlockSpec((ROW_BLK, W_AGG), lambda i: (i, 0)),
        out_shape=jax.ShapeDtypeStruct((N + 8, W_AGG), jnp.float32),
    )(agg_a, agg_b, g2, dinv, b2a, b2b, w3a, w3b)


def _tc_final_nodes(agg, gprev, dinv, b_a):
    grid = (N // ROW_BLK,)

    def body(agg_ref, g_ref, dinv_ref, ba_ref, out_ref):
        s = (agg_ref[0] + agg_ref[1] + g_ref[...]) * dinv_ref[...]
        out_ref[...] = jnp.maximum(s + ba_ref[...], 0.0)

    return pl.pallas_call(
        body,
        grid=grid,
        in_specs=[
            pl.BlockSpec((2, ROW_BLK, W_AGG), lambda i: (0, i, 0)),
            pl.BlockSpec((ROW_BLK, W_AGG), lambda i: (i, 0)),
            pl.BlockSpec((ROW_BLK, 1), lambda i: (i, 0)),
            pl.BlockSpec(b_a.shape, lambda i: (0, 0)),
        ],
        out_specs=pl.BlockSpec((ROW_BLK, W_AGG), lambda i: (i, 0)),
        out_shape=jax.ShapeDtypeStruct((N, W_AGG), jnp.float32),
    )(agg, gprev, dinv, b_a)


def _tc_readout(h3r, wl1p, bl1, gamma, beta, wl2, bl2):
    def body(h_ref, w1_ref, b1_ref, ga_ref, be_ref, w2_ref, b2_ref, out_ref):
        z = jnp.dot(h_ref[...].astype(jnp.bfloat16), w1_ref[...],
                    preferred_element_type=jnp.float32)
        z = z + b1_ref[...]
        mean = jnp.mean(z, axis=0, keepdims=True)
        var = jnp.mean((z - mean) ** 2, axis=0, keepdims=True)
        zn = (z - mean) * lax.rsqrt(var + 1e-5) * ga_ref[...] + be_ref[...]
        zr = jnp.maximum(zn, 0.0)
        o = jnp.dot(zr.astype(jnp.bfloat16), w2_ref[...],
                    preferred_element_type=jnp.float32)
        out_ref[...] = o + b2_ref[...]

    return pl.pallas_call(
        body,
        out_shape=jax.ShapeDtypeStruct((B_GRAPHS, 1), jnp.float32),
    )(h3r, wl1p, bl1, gamma, beta, wl2, bl2)


def kernel(x, edge_index, W1, b1, W2, b2, W3, b3, Wl1, bl1, gamma, beta,
           Wl2, bl2):
    try:
        info = plsc.get_sparse_core_info()
        nc, ns = info.num_cores, info.num_subcores
    except Exception:
        nc, ns = 2, 16
    nw, k_chunks, zchunks, sub_rows, acc_rows = _sc_worker_geom(nc, ns)

    epad = nw * k_chunks * CH - E
    edges3 = jnp.pad(edge_index, ((0, 0), (0, epad)),
                     constant_values=N).reshape(2, nw * k_chunks, CH)

    zeros1 = jnp.zeros((sub_rows,), jnp.float32)
    zeros2 = jnp.zeros((sub_rows, W_AGG), jnp.float32)

    degp = _sc_degree(edges3, zeros1, nc, ns)
    degp = degp.reshape(nc, ns * sub_rows)[:, :N, None]

    bf = jnp.bfloat16
    w1p = jnp.pad(W1, ((0, W_AGG - 5), (0, 0))).astype(bf).astype(jnp.float32)
    w2st = jnp.stack([W2[:, :W_AGG], W2[:, W_AGG:]]).astype(bf)
    w3p = jnp.pad(W3, ((0, 0), (0, W_AGG - 3))).astype(bf)
    w3a, w3b = w3p[:W_AGG], w3p[W_AGG:]
    b2a, b2b = b2[None, :W_AGG], b2[None, W_AGG:]
    b3p = jnp.pad(b3, (0, W_AGG - 3))[None, :]
    xpad = jnp.pad(x, ((0, 0), (0, W_AGG - 5)))

    dinv, g1 = _tc_prep(degp, xpad)

    def run_agg(g):
        return _sc_aggregate(g, edges3, zeros2, nc, ns)

    agg1 = run_agg(g1)
    g2 = _tc_layer12(agg1, g1, dinv, w1p, b1[None, :], w2st)
    agg2a = run_agg(g2[0])
    agg2b = run_agg(g2[1])
    g3 = _tc_layer3(agg2a, agg2b, g2, dinv, b2a, b2b, w3a, w3b)
    agg3 = run_agg(g3)
    h3 = _tc_final_nodes(agg3, g3, dinv, b3p)

    h3r = h3.reshape(B_GRAPHS, N_PMTS * W_AGG)
    wl1p = jnp.pad(Wl1.reshape(N_PMTS, 3, 32),
                   ((0, 0), (0, W_AGG - 3), (0, 0))).reshape(
                       N_PMTS * W_AGG, 32).astype(jnp.bfloat16)
    return _tc_readout(h3r, wl1p, bl1[None, :], gamma[None, :], beta[None, :],
                       Wl2.astype(jnp.bfloat16), bl2[None, :])


# speedup vs baseline: 1.4625x; 1.0082x over previous
import jax
import jax.numpy as jnp
from jax import lax
from jax.experimental import pallas as pl
from jax.experimental.pallas import tpu as pltpu
from jax.experimental.pallas import tpu_sc as plsc

N_PMTS = 2330
B_GRAPHS = 32
N = B_GRAPHS * N_PMTS
E = N * 16
CH = 128
W_AGG = 8
GK = 8
CORE0_SHARE_PCT = 64
ROW_BLK = 1864


def _sc_worker_geom(nc, ns):
    nw = nc * ns
    k = -(-E // (nw * CH))
    k = -(-k // GK) * GK
    zchunks = -(-(N + 1) // (ns * CH))
    sub_rows = zchunks * CH
    acc_rows = ns * sub_rows
    return nw, k, zchunks, sub_rows, acc_rows


def _sc_degree(edges3, zeros1, nc, ns):
    nw, k_chunks, zchunks, sub_rows, acc_rows = _sc_worker_geom(nc, ns)
    mesh = plsc.VectorSubcoreMesh(core_axis_name="c", subcore_axis_name="s",
                                  num_cores=nc, num_subcores=ns)

    def body(e_hbm, z_hbm, out_hbm, dst_idx, ones_v, acc):
        cid = lax.axis_index("c")
        sid = lax.axis_index("s")
        t = cid * ns + sid
        pltpu.sync_copy(e_hbm.at[1, pl.ds(t * k_chunks, k_chunks), :], dst_idx)
        for i in range(CH // 16):
            ones_v[pl.ds(i * 16, 16)] = jnp.ones((16,), jnp.float32)
        base = sid * sub_rows
        pltpu.sync_copy(z_hbm, acc.at[pl.ds(base, sub_rows)])
        plsc.subcore_barrier()

        def scat(j, _):
            pltpu.sync_copy(ones_v, acc.at[dst_idx.at[j]], add=True)
            return 0
        lax.fori_loop(0, k_chunks, scat, 0)
        plsc.subcore_barrier()
        pltpu.sync_copy(acc.at[pl.ds(base, sub_rows)], out_hbm.at[cid, sid])

    return pl.kernel(
        body,
        out_type=jax.ShapeDtypeStruct((nc, ns, sub_rows), jnp.float32),
        mesh=mesh,
        scratch_types=[
            pltpu.VMEM((k_chunks, CH), jnp.int32),
            pltpu.VMEM((CH,), jnp.float32),
            pltpu.VMEM_SHARED((acc_rows,), jnp.float32),
        ],
        compiler_params=pltpu.CompilerParams(use_tc_tiling_on_sc=False),
    )(edges3, zeros1)


def _sc_aggregate(g, edges3, zeros2, nc, ns):
    nw, k_chunks, zchunks, sub_rows, acc_rows = _sc_worker_geom(nc, ns)
    mesh = plsc.VectorSubcoreMesh(core_axis_name="c", subcore_axis_name="s",
                                  num_cores=nc, num_subcores=ns)

    n_groups = k_chunks // GK
    total_groups = 2 * n_groups
    g_a = (total_groups * CORE0_SHARE_PCT + 50) // 100
    g_b = total_groups - g_a
    c_a, c_b = g_a * GK, g_b * GK
    k_max = max(c_a, c_b)

    def body(g_hbm, e_hbm, z_hbm, out_hbm,
             src_idx, dst_idx, bufs, acc, gsem, ssem, isem):
        cid = lax.axis_index("c")
        sid = lax.axis_index("s")
        base = sid * sub_rows
        pltpu.sync_copy(z_hbm, acc.at[pl.ds(base, sub_rows), :])

        def run(start, n_chk, n_grp):
            def iload_sync(grp):
                bk = grp % 3
                pltpu.sync_copy(e_hbm.at[0, pl.ds(start + grp * GK, GK), :],
                                src_idx.at[bk])
                pltpu.sync_copy(e_hbm.at[1, pl.ds(start + grp * GK, GK), :],
                                dst_idx.at[bk])

            def iload_start(grp):
                bk = grp % 3
                pltpu.async_copy(e_hbm.at[0, pl.ds(start + grp * GK, GK), :],
                                 src_idx.at[bk], isem)
                pltpu.async_copy(e_hbm.at[1, pl.ds(start + grp * GK, GK), :],
                                 dst_idx.at[bk], isem)

            def iload_wait(grp):
                bk = grp % 3
                pltpu.make_async_copy(
                    e_hbm.at[0, pl.ds(start + grp * GK, GK), :],
                    src_idx.at[bk], isem).wait()
                pltpu.make_async_copy(
                    e_hbm.at[1, pl.ds(start + grp * GK, GK), :],
                    dst_idx.at[bk], isem).wait()

            def gstart(grp, b):
                pltpu.async_copy(g_hbm.at[src_idx.at[grp % 3, b]],
                                 bufs.at[(grp % 2) * GK + b], gsem)

            def gwait(grp, b):
                pltpu.make_async_copy(g_hbm.at[src_idx.at[grp % 3, b]],
                                      bufs.at[(grp % 2) * GK + b], gsem).wait()

            def sstart(grp, b):
                pltpu.async_copy(bufs.at[(grp % 2) * GK + b],
                                 acc.at[dst_idx.at[grp % 3, b]], ssem,
                                 add=True)

            def swait(grp, b):
                pltpu.make_async_copy(bufs.at[(grp % 2) * GK + b],
                                      acc.at[dst_idx.at[grp % 3, b]],
                                      ssem).wait()

            iload_sync(0)
            for b in range(GK):
                gstart(0, b)

            @pl.when(1 < n_grp)
            def _():
                iload_start(1)

            def loop(m, _):
                for b in range(GK):
                    gwait(m, b)

                @pl.when(m >= 1)
                def _():
                    for b in range(GK):
                        swait(m - 1, b)
                for b in range(GK):
                    sstart(m, b)

                @pl.when(m + 1 < n_grp)
                def _():
                    iload_wait(m + 1)
                    for b in range(GK):
                        gstart(m + 1, b)

                @pl.when(m + 2 < n_grp)
                def _():
                    iload_start(m + 2)
                return 0
            lax.fori_loop(0, n_grp, loop, 0)

            for b in range(GK):
                swait(n_grp - 1, b)

        @pl.when(cid == 0)
        def _():
            run(sid * c_a, c_a, g_a)

        @pl.when(cid == 1)
        def _():
            run(ns * c_a + sid * c_b, c_b, g_b)

        plsc.subcore_barrier()
        pltpu.sync_copy(acc.at[pl.ds(base, sub_rows), :],
                        out_hbm.at[cid, pl.ds(base, sub_rows), :])

    return pl.kernel(
        body,
        out_type=jax.ShapeDtypeStruct((nc, acc_rows, W_AGG), jnp.float32),
        mesh=mesh,
        scratch_types=[
            pltpu.VMEM((3, GK, CH), jnp.int32),
            pltpu.VMEM((3, GK, CH), jnp.int32),
            pltpu.VMEM((2 * GK, CH, W_AGG), jnp.float32),
            pltpu.VMEM_SHARED((acc_rows, W_AGG), jnp.float32),
            pltpu.SemaphoreType.DMA,
            pltpu.SemaphoreType.DMA,
            pltpu.SemaphoreType.DMA,
        ],
        compiler_params=pltpu.CompilerParams(use_tc_tiling_on_sc=False),
    )(g, edges3, zeros2)


def _tc_prep(degp, xpad):
    grid = (N // ROW_BLK,)

    def body(deg_ref, x_ref, dinv_ref, g1_ref):
        d = deg_ref[0] + deg_ref[1] + 1.0
        dv = lax.rsqrt(d)
        dinv_ref[...] = dv
        xb = x_ref[...].astype(jnp.bfloat16).astype(jnp.float32)
        g1_ref[...] = xb * dv

    return pl.pallas_call(
        body,
        grid=grid,
        in_specs=[
            pl.BlockSpec((2, ROW_BLK, 1), lambda i: (0, i, 0)),
            pl.BlockSpec((ROW_BLK, W_AGG), lambda i: (i, 0)),
        ],
        out_specs=[
            pl.BlockSpec((ROW_BLK, 1), lambda i: (i, 0)),
            pl.BlockSpec((ROW_BLK, W_AGG), lambda i: (i, 0)),
        ],
        out_shape=[
            jax.ShapeDtypeStruct((N, 1), jnp.float32),
            jax.ShapeDtypeStruct((N + 8, W_AGG), jnp.float32),
        ],
    )(degp, xpad)


def _tc_layer12(agg, gprev, dinv, w_a, b_a, w_bst):
    grid = (N // ROW_BLK,)

    def body(agg_ref, g_ref, dinv_ref, wa_ref, ba_ref, wb_ref, out_ref):
        dv = dinv_ref[...]
        s = (agg_ref[0] + agg_ref[1] + g_ref[...]) * dv
        h = jnp.dot(s, wa_ref[...], preferred_element_type=jnp.float32, precision=lax.Precision.HIGHEST)
        h = jnp.maximum(h + ba_ref[...], 0.0).astype(jnp.bfloat16)
        out_ref[0] = jnp.dot(h, wb_ref[0], preferred_element_type=jnp.float32) * dv
        out_ref[1] = jnp.dot(h, wb_ref[1], preferred_element_type=jnp.float32) * dv

    return pl.pallas_call(
        body,
        grid=grid,
        in_specs=[
            pl.BlockSpec((2, ROW_BLK, W_AGG), lambda i: (0, i, 0)),
            pl.BlockSpec((ROW_BLK, W_AGG), lambda i: (i, 0)),
            pl.BlockSpec((ROW_BLK, 1), lambda i: (i, 0)),
            pl.BlockSpec(w_a.shape, lambda i: (0, 0)),
            pl.BlockSpec(b_a.shape, lambda i: (0, 0)),
            pl.BlockSpec(w_bst.shape, lambda i: (0, 0, 0)),
        ],
        out_specs=pl.BlockSpec((2, ROW_BLK, W_AGG), lambda i: (0, i, 0)),
        out_shape=jax.ShapeDtypeStruct((2, N + 8, W_AGG), jnp.float32),
    )(agg, gprev, dinv, w_a, b_a, w_bst)


def _tc_layer3(agg_a, agg_b, g2, dinv, b2a, b2b, w3a, w3b):
    grid = (N // ROW_BLK,)

    def body(aa_ref, ab_ref, g_ref, dinv_ref, ba_ref, bb_ref,
             wa_ref, wb_ref, out_ref):
        dv = dinv_ref[...]
        sa = (aa_ref[0] + aa_ref[1] + g_ref[0]) * dv
        sb = (ab_ref[0] + ab_ref[1] + g_ref[1]) * dv
        ha = jnp.maximum(sa + ba_ref[...], 0.0).astype(jnp.bfloat16)
        hb = jnp.maximum(sb + bb_ref[...], 0.0).astype(jnp.bfloat16)
        o = (jnp.dot(ha, wa_ref[...], preferred_element_type=jnp.float32)
             + jnp.dot(hb, wb_ref[...], preferred_element_type=jnp.float32))
        out_ref[...] = o * dv

    return pl.pallas_call(
        body,
        grid=grid,
        in_specs=[
            pl.BlockSpec((2, ROW_BLK, W_AGG), lambda i: (0, i, 0)),
            pl.BlockSpec((2, ROW_BLK, W_AGG), lambda i: (0, i, 0)),
            pl.BlockSpec((2, ROW_BLK, W_AGG), lambda i: (0, i, 0)),
            pl.BlockSpec((ROW_BLK, 1), lambda i: (i, 0)),
            pl.BlockSpec(b2a.shape, lambda i: (0, 0)),
            pl.BlockSpec(b2b.shape, lambda i: (0, 0)),
            pl.BlockSpec(w3a.shape, lambda i: (0, 0)),
            pl.BlockSpec(w3b.shape, lambda i: (0, 0)),
        ],
        out_specs=pl.BlockSpec((ROW_BLK, W_AGG), lambda i: (i, 0)),
        out_shape=jax.ShapeDtypeStruct((N + 8, W_AGG), jnp.float32),
    )(agg_a, agg_b, g2, dinv, b2a, b2b, w3a, w3b)


def _tc_final_nodes(agg, gprev, dinv, b_a):
    grid = (N // ROW_BLK,)

    def body(agg_ref, g_ref, dinv_ref, ba_ref, out_ref):
        s = (agg_ref[0] + agg_ref[1] + g_ref[...]) * dinv_ref[...]
        out_ref[...] = jnp.maximum(s + ba_ref[...], 0.0).astype(jnp.bfloat16)

    return pl.pallas_call(
        body,
        grid=grid,
        in_specs=[
            pl.BlockSpec((2, ROW_BLK, W_AGG), lambda i: (0, i, 0)),
            pl.BlockSpec((ROW_BLK, W_AGG), lambda i: (i, 0)),
            pl.BlockSpec((ROW_BLK, 1), lambda i: (i, 0)),
            pl.BlockSpec(b_a.shape, lambda i: (0, 0)),
        ],
        out_specs=pl.BlockSpec((ROW_BLK, W_AGG), lambda i: (i, 0)),
        out_shape=jax.ShapeDtypeStruct((N, W_AGG), jnp.bfloat16),
    )(agg, gprev, dinv, b_a)


def _tc_readout(h3r, wl1p, bl1, gamma, beta, wl2, bl2):
    def body(h_ref, w1_ref, b1_ref, ga_ref, be_ref, w2_ref, b2_ref, out_ref):
        z = jnp.dot(h_ref[...], w1_ref[...],
                    preferred_element_type=jnp.float32)
        z = z + b1_ref[...]
        mean = jnp.mean(z, axis=0, keepdims=True)
        var = jnp.mean((z - mean) ** 2, axis=0, keepdims=True)
        zn = (z - mean) * lax.rsqrt(var + 1e-5) * ga_ref[...] + be_ref[...]
        zr = jnp.maximum(zn, 0.0)
        o = jnp.dot(zr.astype(jnp.bfloat16), w2_ref[...],
                    preferred_element_type=jnp.float32)
        out_ref[...] = o + b2_ref[...]

    return pl.pallas_call(
        body,
        out_shape=jax.ShapeDtypeStruct((B_GRAPHS, 1), jnp.float32),
    )(h3r, wl1p, bl1, gamma, beta, wl2, bl2)


def kernel(x, edge_index, W1, b1, W2, b2, W3, b3, Wl1, bl1, gamma, beta,
           Wl2, bl2):
    try:
        info = plsc.get_sparse_core_info()
        nc, ns = info.num_cores, info.num_subcores
    except Exception:
        nc, ns = 2, 16
    nw, k_chunks, zchunks, sub_rows, acc_rows = _sc_worker_geom(nc, ns)

    epad = nw * k_chunks * CH - E
    edges3 = jnp.pad(edge_index, ((0, 0), (0, epad)),
                     constant_values=N).reshape(2, nw * k_chunks, CH)

    zeros1 = jnp.zeros((sub_rows,), jnp.float32)
    zeros2 = jnp.zeros((sub_rows, W_AGG), jnp.float32)

    degp = _sc_degree(edges3, zeros1, nc, ns)
    degp = degp.reshape(nc, ns * sub_rows)[:, :N, None]

    bf = jnp.bfloat16
    w1p = jnp.pad(W1, ((0, W_AGG - 5), (0, 0))).astype(bf).astype(jnp.float32)
    w2st = jnp.stack([W2[:, :W_AGG], W2[:, W_AGG:]]).astype(bf)
    w3p = jnp.pad(W3, ((0, 0), (0, W_AGG - 3))).astype(bf)
    w3a, w3b = w3p[:W_AGG], w3p[W_AGG:]
    b2a, b2b = b2[None, :W_AGG], b2[None, W_AGG:]
    b3p = jnp.pad(b3, (0, W_AGG - 3))[None, :]
    xpad = jnp.pad(x, ((0, 0), (0, W_AGG - 5)))

    dinv, g1 = _tc_prep(degp, xpad)

    def run_agg(g):
        return _sc_aggregate(g, edges3, zeros2, nc, ns)

    agg1 = run_agg(g1)
    g2 = _tc_layer12(agg1, g1, dinv, w1p, b1[None, :], w2st)
    agg2a = run_agg(g2[0])
    agg2b = run_agg(g2[1])
    g3 = _tc_layer3(agg2a, agg2b, g2, dinv, b2a, b2b, w3a, w3b)
    agg3 = run_agg(g3)
    h3 = _tc_final_nodes(agg3, g3, dinv, b3p)

    h3r = h3.reshape(B_GRAPHS, N_PMTS * W_AGG)
    wl1p = jnp.pad(Wl1.reshape(N_PMTS, 3, 32),
                   ((0, 0), (0, W_AGG - 3), (0, 0))).reshape(
                       N_PMTS * W_AGG, 32).astype(jnp.bfloat16)
    return _tc_readout(h3r, wl1p, bl1[None, :], gamma[None, :], beta[None, :],
                       Wl2.astype(jnp.bfloat16), bl2[None, :])


# speedup vs baseline: 1.4968x; 1.0234x over previous
import jax
import jax.numpy as jnp
from jax import lax
from jax.experimental import pallas as pl
from jax.experimental.pallas import tpu as pltpu
from jax.experimental.pallas import tpu_sc as plsc

N_PMTS = 2330
B_GRAPHS = 32
N = B_GRAPHS * N_PMTS
E = N * 16
CH = 128
W_AGG = 8
GK = 8
CORE0_SHARE_PCT = 64
ROW_BLK = 3728


def _sc_worker_geom(nc, ns):
    nw = nc * ns
    k = -(-E // (nw * CH))
    k = -(-k // GK) * GK
    zchunks = -(-(N + 1) // (ns * CH))
    sub_rows = zchunks * CH
    acc_rows = ns * sub_rows
    return nw, k, zchunks, sub_rows, acc_rows


def _sc_degree(edges3, zeros1, nc, ns):
    nw, k_chunks, zchunks, sub_rows, acc_rows = _sc_worker_geom(nc, ns)
    mesh = plsc.VectorSubcoreMesh(core_axis_name="c", subcore_axis_name="s",
                                  num_cores=nc, num_subcores=ns)

    def body(e_hbm, z_hbm, out_hbm, dst_idx, ones_v, acc):
        cid = lax.axis_index("c")
        sid = lax.axis_index("s")
        t = cid * ns + sid
        pltpu.sync_copy(e_hbm.at[1, pl.ds(t * k_chunks, k_chunks), :], dst_idx)
        for i in range(CH // 16):
            ones_v[pl.ds(i * 16, 16)] = jnp.ones((16,), jnp.float32)
        base = sid * sub_rows
        pltpu.sync_copy(z_hbm, acc.at[pl.ds(base, sub_rows)])
        plsc.subcore_barrier()

        def scat(j, _):
            pltpu.sync_copy(ones_v, acc.at[dst_idx.at[j]], add=True)
            return 0
        lax.fori_loop(0, k_chunks, scat, 0)
        plsc.subcore_barrier()
        pltpu.sync_copy(acc.at[pl.ds(base, sub_rows)], out_hbm.at[cid, sid])

    return pl.kernel(
        body,
        out_type=jax.ShapeDtypeStruct((nc, ns, sub_rows), jnp.float32),
        mesh=mesh,
        scratch_types=[
            pltpu.VMEM((k_chunks, CH), jnp.int32),
            pltpu.VMEM((CH,), jnp.float32),
            pltpu.VMEM_SHARED((acc_rows,), jnp.float32),
        ],
        compiler_params=pltpu.CompilerParams(use_tc_tiling_on_sc=False),
    )(edges3, zeros1)


def _sc_aggregate(g, edges3, zeros2, nc, ns):
    nw, k_chunks, zchunks, sub_rows, acc_rows = _sc_worker_geom(nc, ns)
    mesh = plsc.VectorSubcoreMesh(core_axis_name="c", subcore_axis_name="s",
                                  num_cores=nc, num_subcores=ns)

    n_groups = k_chunks // GK
    total_groups = 2 * n_groups
    g_a = (total_groups * CORE0_SHARE_PCT + 50) // 100
    g_b = total_groups - g_a
    c_a, c_b = g_a * GK, g_b * GK
    k_max = max(c_a, c_b)

    def body(g_hbm, e_hbm, z_hbm, out_hbm,
             src_idx, dst_idx, bufs, acc, gsem, ssem, isem):
        cid = lax.axis_index("c")
        sid = lax.axis_index("s")
        base = sid * sub_rows
        pltpu.sync_copy(z_hbm, acc.at[pl.ds(base, sub_rows), :])

        def run(start, n_chk, n_grp):
            def iload_sync(grp):
                bk = grp % 3
                pltpu.sync_copy(e_hbm.at[0, pl.ds(start + grp * GK, GK), :],
                                src_idx.at[bk])
                pltpu.sync_copy(e_hbm.at[1, pl.ds(start + grp * GK, GK), :],
                                dst_idx.at[bk])

            def iload_start(grp):
                bk = grp % 3
                pltpu.async_copy(e_hbm.at[0, pl.ds(start + grp * GK, GK), :],
                                 src_idx.at[bk], isem)
                pltpu.async_copy(e_hbm.at[1, pl.ds(start + grp * GK, GK), :],
                                 dst_idx.at[bk], isem)

            def iload_wait(grp):
                bk = grp % 3
                pltpu.make_async_copy(
                    e_hbm.at[0, pl.ds(start + grp * GK, GK), :],
                    src_idx.at[bk], isem).wait()
                pltpu.make_async_copy(
                    e_hbm.at[1, pl.ds(start + grp * GK, GK), :],
                    dst_idx.at[bk], isem).wait()

            def gstart(grp, b):
                pltpu.async_copy(g_hbm.at[src_idx.at[grp % 3, b]],
                                 bufs.at[(grp % 2) * GK + b], gsem)

            def gwait(grp, b):
                pltpu.make_async_copy(g_hbm.at[src_idx.at[grp % 3, b]],
                                      bufs.at[(grp % 2) * GK + b], gsem).wait()

            def sstart(grp, b):
                pltpu.async_copy(bufs.at[(grp % 2) * GK + b],
                                 acc.at[dst_idx.at[grp % 3, b]], ssem,
                                 add=True)

            def swait(grp, b):
                pltpu.make_async_copy(bufs.at[(grp % 2) * GK + b],
                                      acc.at[dst_idx.at[grp % 3, b]],
                                      ssem).wait()

            iload_sync(0)
            for b in range(GK):
                gstart(0, b)

            @pl.when(1 < n_grp)
            def _():
                iload_start(1)

            def loop(m, _):
                for b in range(GK):
                    gwait(m, b)

                @pl.when(m >= 1)
                def _():
                    for b in range(GK):
                        swait(m - 1, b)
                for b in range(GK):
                    sstart(m, b)

                @pl.when(m + 1 < n_grp)
                def _():
                    iload_wait(m + 1)
                    for b in range(GK):
                        gstart(m + 1, b)

                @pl.when(m + 2 < n_grp)
                def _():
                    iload_start(m + 2)
                return 0
            lax.fori_loop(0, n_grp, loop, 0)

            for b in range(GK):
                swait(n_grp - 1, b)

        @pl.when(cid == 0)
        def _():
            run(sid * c_a, c_a, g_a)

        @pl.when(cid == 1)
        def _():
            run(ns * c_a + sid * c_b, c_b, g_b)

        plsc.subcore_barrier()
        pltpu.sync_copy(acc.at[pl.ds(base, sub_rows), :],
                        out_hbm.at[cid, pl.ds(base, sub_rows), :])

    return pl.kernel(
        body,
        out_type=jax.ShapeDtypeStruct((nc, acc_rows, W_AGG), jnp.float32),
        mesh=mesh,
        scratch_types=[
            pltpu.VMEM((3, GK, CH), jnp.int32),
            pltpu.VMEM((3, GK, CH), jnp.int32),
            pltpu.VMEM((2 * GK, CH, W_AGG), jnp.float32),
            pltpu.VMEM_SHARED((acc_rows, W_AGG), jnp.float32),
            pltpu.SemaphoreType.DMA,
            pltpu.SemaphoreType.DMA,
            pltpu.SemaphoreType.DMA,
        ],
        compiler_params=pltpu.CompilerParams(use_tc_tiling_on_sc=False),
    )(g, edges3, zeros2)


def _tc_prep(degp, xpad):
    grid = (N // ROW_BLK,)

    def body(deg_ref, x_ref, dinv_ref, g1_ref):
        d = deg_ref[0] + deg_ref[1] + 1.0
        dv = lax.rsqrt(d)
        dinv_ref[...] = dv
        xb = x_ref[...].astype(jnp.bfloat16).astype(jnp.float32)
        g1_ref[...] = xb * dv

    return pl.pallas_call(
        body,
        grid=grid,
        in_specs=[
            pl.BlockSpec((2, ROW_BLK, 1), lambda i: (0, i, 0)),
            pl.BlockSpec((ROW_BLK, W_AGG), lambda i: (i, 0)),
        ],
        out_specs=[
            pl.BlockSpec((ROW_BLK, 1), lambda i: (i, 0)),
            pl.BlockSpec((ROW_BLK, W_AGG), lambda i: (i, 0)),
        ],
        out_shape=[
            jax.ShapeDtypeStruct((N, 1), jnp.float32),
            jax.ShapeDtypeStruct((N + 8, W_AGG), jnp.float32),
        ],
    )(degp, xpad)


def _tc_layer12(agg, gprev, dinv, w_a, b_a, w_bst):
    grid = (N // ROW_BLK,)

    def body(agg_ref, g_ref, dinv_ref, wa_ref, ba_ref, wb_ref, out_ref):
        dv = dinv_ref[...]
        s = (agg_ref[0] + agg_ref[1] + g_ref[...]) * dv
        h = jnp.dot(s, wa_ref[...], preferred_element_type=jnp.float32, precision=lax.Precision.HIGHEST)
        h = jnp.maximum(h + ba_ref[...], 0.0).astype(jnp.bfloat16)
        out_ref[0] = jnp.dot(h, wb_ref[0], preferred_element_type=jnp.float32) * dv
        out_ref[1] = jnp.dot(h, wb_ref[1], preferred_element_type=jnp.float32) * dv

    return pl.pallas_call(
        body,
        grid=grid,
        in_specs=[
            pl.BlockSpec((2, ROW_BLK, W_AGG), lambda i: (0, i, 0)),
            pl.BlockSpec((ROW_BLK, W_AGG), lambda i: (i, 0)),
            pl.BlockSpec((ROW_BLK, 1), lambda i: (i, 0)),
            pl.BlockSpec(w_a.shape, lambda i: (0, 0)),
            pl.BlockSpec(b_a.shape, lambda i: (0, 0)),
            pl.BlockSpec(w_bst.shape, lambda i: (0, 0, 0)),
        ],
        out_specs=pl.BlockSpec((2, ROW_BLK, W_AGG), lambda i: (0, i, 0)),
        out_shape=jax.ShapeDtypeStruct((2, N + 8, W_AGG), jnp.float32),
    )(agg, gprev, dinv, w_a, b_a, w_bst)


def _tc_layer3(agg_a, agg_b, g2, dinv, b2a, b2b, w3a, w3b):
    grid = (N // ROW_BLK,)

    def body(aa_ref, ab_ref, g_ref, dinv_ref, ba_ref, bb_ref,
             wa_ref, wb_ref, out_ref):
        dv = dinv_ref[...]
        sa = (aa_ref[0] + aa_ref[1] + g_ref[0]) * dv
        sb = (ab_ref[0] + ab_ref[1] + g_ref[1]) * dv
        ha = jnp.maximum(sa + ba_ref[...], 0.0).astype(jnp.bfloat16)
        hb = jnp.maximum(sb + bb_ref[...], 0.0).astype(jnp.bfloat16)
        o = (jnp.dot(ha, wa_ref[...], preferred_element_type=jnp.float32)
             + jnp.dot(hb, wb_ref[...], preferred_element_type=jnp.float32))
        out_ref[...] = o * dv

    return pl.pallas_call(
        body,
        grid=grid,
        in_specs=[
            pl.BlockSpec((2, ROW_BLK, W_AGG), lambda i: (0, i, 0)),
            pl.BlockSpec((2, ROW_BLK, W_AGG), lambda i: (0, i, 0)),
            pl.BlockSpec((2, ROW_BLK, W_AGG), lambda i: (0, i, 0)),
            pl.BlockSpec((ROW_BLK, 1), lambda i: (i, 0)),
            pl.BlockSpec(b2a.shape, lambda i: (0, 0)),
            pl.BlockSpec(b2b.shape, lambda i: (0, 0)),
            pl.BlockSpec(w3a.shape, lambda i: (0, 0)),
            pl.BlockSpec(w3b.shape, lambda i: (0, 0)),
        ],
        out_specs=pl.BlockSpec((ROW_BLK, W_AGG), lambda i: (i, 0)),
        out_shape=jax.ShapeDtypeStruct((N + 8, W_AGG), jnp.float32),
    )(agg_a, agg_b, g2, dinv, b2a, b2b, w3a, w3b)


def _tc_final_nodes(agg, gprev, dinv, b_a):
    grid = (N // ROW_BLK,)

    def body(agg_ref, g_ref, dinv_ref, ba_ref, out_ref):
        s = (agg_ref[0] + agg_ref[1] + g_ref[...]) * dinv_ref[...]
        out_ref[...] = jnp.maximum(s + ba_ref[...], 0.0).astype(jnp.bfloat16)

    return pl.pallas_call(
        body,
        grid=grid,
        in_specs=[
            pl.BlockSpec((2, ROW_BLK, W_AGG), lambda i: (0, i, 0)),
            pl.BlockSpec((ROW_BLK, W_AGG), lambda i: (i, 0)),
            pl.BlockSpec((ROW_BLK, 1), lambda i: (i, 0)),
            pl.BlockSpec(b_a.shape, lambda i: (0, 0)),
        ],
        out_specs=pl.BlockSpec((ROW_BLK, W_AGG), lambda i: (i, 0)),
        out_shape=jax.ShapeDtypeStruct((N, W_AGG), jnp.bfloat16),
    )(agg, gprev, dinv, b_a)


def _tc_readout(h3r, wl1p, bl1, gamma, beta, wl2, bl2):
    def body(h_ref, w1_ref, b1_ref, ga_ref, be_ref, w2_ref, b2_ref, out_ref):
        z = jnp.dot(h_ref[...], w1_ref[...],
                    preferred_element_type=jnp.float32)
        z = z + b1_ref[...]
        mean = jnp.mean(z, axis=0, keepdims=True)
        var = jnp.mean((z - mean) ** 2, axis=0, keepdims=True)
        zn = (z - mean) * lax.rsqrt(var + 1e-5) * ga_ref[...] + be_ref[...]
        zr = jnp.maximum(zn, 0.0)
        o = jnp.dot(zr.astype(jnp.bfloat16), w2_ref[...],
                    preferred_element_type=jnp.float32)
        out_ref[...] = o + b2_ref[...]

    return pl.pallas_call(
        body,
        out_shape=jax.ShapeDtypeStruct((B_GRAPHS, 1), jnp.float32),
    )(h3r, wl1p, bl1, gamma, beta, wl2, bl2)


def kernel(x, edge_index, W1, b1, W2, b2, W3, b3, Wl1, bl1, gamma, beta,
           Wl2, bl2):
    try:
        info = plsc.get_sparse_core_info()
        nc, ns = info.num_cores, info.num_subcores
    except Exception:
        nc, ns = 2, 16
    nw, k_chunks, zchunks, sub_rows, acc_rows = _sc_worker_geom(nc, ns)

    epad = nw * k_chunks * CH - E
    edges3 = jnp.pad(edge_index, ((0, 0), (0, epad)),
                     constant_values=N).reshape(2, nw * k_chunks, CH)

    zeros1 = jnp.zeros((sub_rows,), jnp.float32)
    zeros2 = jnp.zeros((sub_rows, W_AGG), jnp.float32)

    degp = _sc_degree(edges3, zeros1, nc, ns)
    degp = degp.reshape(nc, ns * sub_rows)[:, :N, None]

    bf = jnp.bfloat16
    w1p = jnp.pad(W1, ((0, W_AGG - 5), (0, 0))).astype(bf).astype(jnp.float32)
    w2st = jnp.stack([W2[:, :W_AGG], W2[:, W_AGG:]]).astype(bf)
    w3p = jnp.pad(W3, ((0, 0), (0, W_AGG - 3))).astype(bf)
    w3a, w3b = w3p[:W_AGG], w3p[W_AGG:]
    b2a, b2b = b2[None, :W_AGG], b2[None, W_AGG:]
    b3p = jnp.pad(b3, (0, W_AGG - 3))[None, :]
    xpad = jnp.pad(x, ((0, 0), (0, W_AGG - 5)))

    dinv, g1 = _tc_prep(degp, xpad)

    def run_agg(g):
        return _sc_aggregate(g, edges3, zeros2, nc, ns)

    agg1 = run_agg(g1)
    g2 = _tc_layer12(agg1, g1, dinv, w1p, b1[None, :], w2st)
    agg2a = run_agg(g2[0])
    agg2b = run_agg(g2[1])
    g3 = _tc_layer3(agg2a, agg2b, g2, dinv, b2a, b2b, w3a, w3b)
    agg3 = run_agg(g3)
    h3 = _tc_final_nodes(agg3, g3, dinv, b3p)

    h3r = h3.reshape(B_GRAPHS, N_PMTS * W_AGG)
    wl1p = jnp.pad(Wl1.reshape(N_PMTS, 3, 32),
                   ((0, 0), (0, W_AGG - 3), (0, 0))).reshape(
                       N_PMTS * W_AGG, 32).astype(jnp.bfloat16)
    return _tc_readout(h3r, wl1p, bl1[None, :], gamma[None, :], beta[None, :],
                       Wl2.astype(jnp.bfloat16), bl2[None, :])
